# baseline (speedup 1.0000x reference)
import functools

import jax
import jax.numpy as jnp
from jax import lax
from jax.experimental import pallas as pl
from jax.experimental.pallas import tpu as pltpu
from jax.experimental.pallas import tpu_sc as plsc

VOCAB = 1000
D = 1000
D_PAD = 1024
B, L = 1024, 50
N = B * L

NC, NS = 2, 16
NW = NC * NS
B_PER_W = N // NW
CA = 32
CB = 24
PAIR = CA + CB
NPAIR = 28
TAIL_OFF = NPAIR * PAIR


def _emb_body(idx_hbm, table_hbm, out_hbm, table_sp, idx_v, rows0, rows1,
              sem0, sem1):
    cid = lax.axis_index("c")
    sid = lax.axis_index("s")
    wid = sid * NC + cid
    base = wid * B_PER_W

    @pl.when(sid == 0)
    def _():
        pltpu.sync_copy(table_hbm, table_sp)

    pltpu.sync_copy(idx_hbm.at[pl.ds(base, B_PER_W)], idx_v)
    plsc.subcore_barrier()

    def gather_start(off, n, rows, sem):
        pltpu.async_copy(table_sp.at[idx_v.at[pl.ds(off, n)]], rows, sem)

    def gather_wait(off, n, rows, sem):
        pltpu.make_async_copy(table_sp.at[idx_v.at[pl.ds(off, n)]],
                              rows, sem).wait()

    def write_out(off, n, rows):
        pltpu.sync_copy(rows, out_hbm.at[pl.ds(base + off, n)])

    gather_start(0, CA, rows0, sem0)
    gather_start(CA, CB, rows1, sem1)

    def body(j, carry):
        off = j * PAIR
        gather_wait(off, CA, rows0, sem0)
        write_out(off, CA, rows0)
        gather_start(off + PAIR, CA, rows0, sem0)
        gather_wait(off + CA, CB, rows1, sem1)
        write_out(off + CA, CB, rows1)
        gather_start(off + PAIR + CA, CB, rows1, sem1)
        return carry

    lax.fori_loop(0, NPAIR - 1, body, 0)

    off = (NPAIR - 1) * PAIR
    gather_wait(off, CA, rows0, sem0)
    write_out(off, CA, rows0)
    gather_start(TAIL_OFF, CA, rows0, sem0)
    gather_wait(off + CA, CB, rows1, sem1)
    write_out(off + CA, CB, rows1)
    gather_wait(TAIL_OFF, CA, rows0, sem0)
    write_out(TAIL_OFF, CA, rows0)


_emb = functools.partial(
    pl.kernel,
    out_type=jax.ShapeDtypeStruct((N, 8, 128), jnp.float32),
    mesh=plsc.VectorSubcoreMesh(core_axis_name="c", subcore_axis_name="s",
                                num_cores=NC, num_subcores=NS),
    scratch_types=[
        pltpu.VMEM_SHARED((VOCAB, 8, 128), jnp.float32),
        pltpu.VMEM((B_PER_W,), jnp.int32),
        pltpu.VMEM((CA, 8, 128), jnp.float32),
        pltpu.VMEM((CB, 8, 128), jnp.float32),
        pltpu.SemaphoreType.DMA,
        pltpu.SemaphoreType.DMA,
    ],
)(_emb_body)


@jax.jit
def kernel(x, table):
    idx = x.reshape(-1).astype(jnp.int32)
    table_p = jnp.pad(table, ((0, 0), (0, D_PAD - D))).reshape(VOCAB, 8, 128)
    out = _emb(idx, table_p)
    head = out[:, :7, :].reshape(N, 896)
    tail = out[:, 7, :104]
    return jnp.concatenate([head, tail], axis=1).reshape(B, L, VOCAB)


# speedup vs baseline: 1.4519x; 1.4519x over previous
import functools

import jax
import jax.numpy as jnp
from jax import lax
from jax.experimental import pallas as pl
from jax.experimental.pallas import tpu as pltpu
from jax.experimental.pallas import tpu_sc as plsc

VOCAB = 1000
D = 1000
D_PAD = 1024
B, L = 1024, 50
N = B * L

NSPLIT = 2
NH = N // NSPLIT
BH = B // NSPLIT

NC, NS = 2, 16
NW = NC * NS
B_PER_W = NH // NW
CHUNK = 40
NCHUNK = B_PER_W // CHUNK


def _emb_body(idx_hbm, table_hbm, out_hbm, idx_v, rows0, rows1, sem0, sem1):
    cid = lax.axis_index("c")
    sid = lax.axis_index("s")
    wid = sid * NC + cid
    base = wid * B_PER_W

    pltpu.sync_copy(idx_hbm.at[pl.ds(base, B_PER_W)], idx_v)

    def gather_start(g, rows, sem):
        pltpu.async_copy(table_hbm.at[idx_v.at[pl.ds(g * CHUNK, CHUNK)]],
                         rows, sem)

    def gather_wait(g, rows, sem):
        pltpu.make_async_copy(table_hbm.at[idx_v.at[pl.ds(g * CHUNK, CHUNK)]],
                              rows, sem).wait()

    def write_out(g, rows):
        pltpu.sync_copy(rows, out_hbm.at[pl.ds(base + g * CHUNK, CHUNK)])

    gather_start(0, rows0, sem0)
    gather_start(1, rows1, sem1)

    def body(j, carry):
        g = j * 2
        gather_wait(g, rows0, sem0)
        write_out(g, rows0)
        gather_start(g + 2, rows0, sem0)
        gather_wait(g + 1, rows1, sem1)
        write_out(g + 1, rows1)
        gather_start(g + 3, rows1, sem1)
        return carry

    lax.fori_loop(0, NCHUNK // 2 - 1, body, 0)

    g = NCHUNK - 2
    gather_wait(g, rows0, sem0)
    write_out(g, rows0)
    gather_wait(g + 1, rows1, sem1)
    write_out(g + 1, rows1)


_emb = functools.partial(
    pl.kernel,
    out_type=jax.ShapeDtypeStruct((NH, D_PAD), jnp.float32),
    mesh=plsc.VectorSubcoreMesh(core_axis_name="c", subcore_axis_name="s",
                                num_cores=NC, num_subcores=NS),
    scratch_types=[
        pltpu.VMEM((B_PER_W,), jnp.int32),
        pltpu.VMEM((CHUNK, D_PAD), jnp.float32),
        pltpu.VMEM((CHUNK, D_PAD), jnp.float32),
        pltpu.SemaphoreType.DMA,
        pltpu.SemaphoreType.DMA,
    ],
)(_emb_body)


@jax.jit
def kernel(x, table):
    idx = x.reshape(-1).astype(jnp.int32)
    table_p = jnp.pad(table, ((0, 0), (0, D_PAD - D)))
    halves = [
        _emb(idx[h * NH:(h + 1) * NH], table_p)[:, :D].reshape(BH, L, VOCAB)
        for h in range(NSPLIT)
    ]
    return jnp.concatenate(halves, axis=0)


# speedup vs baseline: 1.5906x; 1.0955x over previous
import functools

import jax
import jax.numpy as jnp
from jax import lax
from jax.experimental import pallas as pl
from jax.experimental.pallas import tpu as pltpu
from jax.experimental.pallas import tpu_sc as plsc

VOCAB = 1000
D = 1000
D_PAD = 1024
B, L = 1024, 50
N = B * L

NC, NS = 2, 16
NW = NC * NS
B_PER_W = N // NW
CA = 32
CB = 24
PAIR = CA + CB
NPAIR = 28
TAIL_OFF = NPAIR * PAIR

GB = 4
GR = GB * L
GRID = B // GB


def _emb_body(idx_hbm, table_hbm, out_hbm, table_sp, idx_v, rows0, rows1,
              sem0, sem1):
    cid = lax.axis_index("c")
    sid = lax.axis_index("s")
    wid = sid * NC + cid
    base = wid * B_PER_W

    @pl.when(sid == 0)
    def _():
        pltpu.sync_copy(table_hbm, table_sp)

    pltpu.sync_copy(idx_hbm.at[pl.ds(base, B_PER_W)], idx_v)
    plsc.subcore_barrier()

    def gather_start(off, n, rows, sem):
        pltpu.async_copy(table_sp.at[idx_v.at[pl.ds(off, n)]], rows, sem)

    def gather_wait(off, n, rows, sem):
        pltpu.make_async_copy(table_sp.at[idx_v.at[pl.ds(off, n)]],
                              rows, sem).wait()

    def write_out(off, n, rows):
        pltpu.sync_copy(rows, out_hbm.at[pl.ds(base + off, n)])

    gather_start(0, CA, rows0, sem0)
    gather_start(CA, CB, rows1, sem1)

    def body(j, carry):
        off = j * PAIR
        gather_wait(off, CA, rows0, sem0)
        write_out(off, CA, rows0)
        gather_start(off + PAIR, CA, rows0, sem0)
        gather_wait(off + CA, CB, rows1, sem1)
        write_out(off + CA, CB, rows1)
        gather_start(off + PAIR + CA, CB, rows1, sem1)
        return carry

    lax.fori_loop(0, NPAIR - 1, body, 0)

    off = (NPAIR - 1) * PAIR
    gather_wait(off, CA, rows0, sem0)
    write_out(off, CA, rows0)
    gather_start(TAIL_OFF, CA, rows0, sem0)
    gather_wait(off + CA, CB, rows1, sem1)
    write_out(off + CA, CB, rows1)
    gather_wait(TAIL_OFF, CA, rows0, sem0)
    write_out(TAIL_OFF, CA, rows0)


_emb = functools.partial(
    pl.kernel,
    out_type=jax.ShapeDtypeStruct((N, 8, 128), jnp.float32),
    mesh=plsc.VectorSubcoreMesh(core_axis_name="c", subcore_axis_name="s",
                                num_cores=NC, num_subcores=NS),
    scratch_types=[
        pltpu.VMEM_SHARED((VOCAB, 8, 128), jnp.float32),
        pltpu.VMEM((B_PER_W,), jnp.int32),
        pltpu.VMEM((CA, 8, 128), jnp.float32),
        pltpu.VMEM((CB, 8, 128), jnp.float32),
        pltpu.SemaphoreType.DMA,
        pltpu.SemaphoreType.DMA,
    ],
)(_emb_body)


def _convert_body(a_ref, o_ref):
    rows = a_ref[...].reshape(GR, D_PAD)
    o_ref[...] = rows[:, :D].reshape(GB, L, D)


_convert = pl.pallas_call(
    _convert_body,
    grid=(GRID,),
    in_specs=[pl.BlockSpec((GR, 8, 128), lambda g: (g, 0, 0))],
    out_specs=pl.BlockSpec((GB, L, D), lambda g: (g, 0, 0)),
    out_shape=jax.ShapeDtypeStruct((B, L, VOCAB), jnp.float32),
)


@jax.jit
def kernel(x, table):
    idx = x.reshape(-1).astype(jnp.int32)
    table_p = jnp.pad(table, ((0, 0), (0, D_PAD - D))).reshape(VOCAB, 8, 128)
    return _convert(_emb(idx, table_p))


# speedup vs baseline: 1.8942x; 1.1909x over previous
import functools

import jax
import jax.numpy as jnp
from jax import lax
from jax.experimental import pallas as pl
from jax.experimental.pallas import tpu as pltpu
from jax.experimental.pallas import tpu_sc as plsc

VOCAB = 1000
D = 1000
D_PAD = 1024
B, L = 1024, 50
N = B * L

NC, NS = 2, 16
NW = NC * NS
B_PER_W = N // NW
CA = 32
CB = 24
PAIR = CA + CB
NPAIR = 28
TAIL_OFF = NPAIR * PAIR

GB = 4
GR = GB * L
GRID = B // GB


def _emb_body(idx_hbm, table_hbm, out_hbm, table_sp, idx_v, rows0, rows1,
              sem0, sem1):
    cid = lax.axis_index("c")
    sid = lax.axis_index("s")
    wid = sid * NC + cid
    base = wid * B_PER_W

    @pl.when(sid == 0)
    def _():
        pltpu.sync_copy(table_hbm, table_sp)

    pltpu.sync_copy(idx_hbm.at[pl.ds(base, B_PER_W)], idx_v)
    plsc.subcore_barrier()

    def gather_start(off, n, rows, sem):
        pltpu.async_copy(table_sp.at[idx_v.at[pl.ds(off, n)]], rows, sem)

    def gather_wait(off, n, rows, sem):
        pltpu.make_async_copy(table_sp.at[idx_v.at[pl.ds(off, n)]],
                              rows, sem).wait()

    def write_out(off, n, rows):
        pltpu.sync_copy(rows, out_hbm.at[pl.ds(base + off, n)])

    gather_start(0, CA, rows0, sem0)
    gather_start(CA, CB, rows1, sem1)

    def body(j, carry):
        off = j * PAIR
        gather_wait(off, CA, rows0, sem0)
        write_out(off, CA, rows0)
        gather_start(off + PAIR, CA, rows0, sem0)
        gather_wait(off + CA, CB, rows1, sem1)
        write_out(off + CA, CB, rows1)
        gather_start(off + PAIR + CA, CB, rows1, sem1)
        return carry

    lax.fori_loop(0, NPAIR - 1, body, 0)

    off = (NPAIR - 1) * PAIR
    gather_wait(off, CA, rows0, sem0)
    write_out(off, CA, rows0)
    gather_start(TAIL_OFF, CA, rows0, sem0)
    gather_wait(off + CA, CB, rows1, sem1)
    write_out(off + CA, CB, rows1)
    gather_wait(TAIL_OFF, CA, rows0, sem0)
    write_out(TAIL_OFF, CA, rows0)


_emb = functools.partial(
    pl.kernel,
    out_type=jax.ShapeDtypeStruct((N, 8, 128), jnp.float32),
    mesh=plsc.VectorSubcoreMesh(core_axis_name="c", subcore_axis_name="s",
                                num_cores=NC, num_subcores=NS),
    scratch_types=[
        pltpu.VMEM_SHARED((VOCAB, 8, 128), jnp.float32),
        pltpu.VMEM((B_PER_W,), jnp.int32),
        pltpu.VMEM((CA, 8, 128), jnp.float32),
        pltpu.VMEM((CB, 8, 128), jnp.float32),
        pltpu.SemaphoreType.DMA,
        pltpu.SemaphoreType.DMA,
    ],
)(_emb_body)


@jax.jit
def kernel(x, table):
    idx = x.reshape(-1).astype(jnp.int32)
    table_p = jnp.pad(table, ((0, 0), (0, D_PAD - D))).reshape(VOCAB, 8, 128)
    out = _emb(idx, table_p)
    return out.reshape(B, L, D_PAD)[:, :, :D]


# speedup vs baseline: 1.8962x; 1.0010x over previous
import functools

import jax
import jax.numpy as jnp
from jax import lax
from jax.experimental import pallas as pl
from jax.experimental.pallas import tpu as pltpu
from jax.experimental.pallas import tpu_sc as plsc

VOCAB = 1000
D = 1000
D_PAD = 1024
B, L = 1024, 50
N = B * L

NC, NS = 2, 16
NW = NC * NS
B_PER_W = N // NW
CA = 32
CB = 24
PAIR = CA + CB
NPAIR = 28
TAIL_OFF = NPAIR * PAIR


def _emb_body(idx_hbm, table_hbm, out_hbm, table_sp, idx_v, rows0, rows1,
              sem0, sem1):
    cid = lax.axis_index("c")
    sid = lax.axis_index("s")
    wid = sid * NC + cid
    base = wid * B_PER_W

    @pl.when(sid == 0)
    def _():
        pltpu.sync_copy(table_hbm, table_sp)

    pltpu.sync_copy(idx_hbm.at[pl.ds(base, B_PER_W)], idx_v)
    plsc.subcore_barrier()

    def gather_start(off, n, rows, sem):
        pltpu.async_copy(table_sp.at[idx_v.at[pl.ds(off, n)]], rows, sem)

    def gather_wait(off, n, rows, sem):
        pltpu.make_async_copy(table_sp.at[idx_v.at[pl.ds(off, n)]],
                              rows, sem).wait()

    def write_out(off, n, rows):
        pltpu.sync_copy(rows, out_hbm.at[pl.ds(base + off, n)])

    gather_start(0, CA, rows0, sem0)
    gather_start(CA, CB, rows1, sem1)

    def body(j, carry):
        off = j * PAIR
        gather_wait(off, CA, rows0, sem0)
        write_out(off, CA, rows0)
        gather_start(off + PAIR, CA, rows0, sem0)
        gather_wait(off + CA, CB, rows1, sem1)
        write_out(off + CA, CB, rows1)
        gather_start(off + PAIR + CA, CB, rows1, sem1)
        return carry

    lax.fori_loop(0, NPAIR - 1, body, 0)

    off = (NPAIR - 1) * PAIR
    gather_wait(off, CA, rows0, sem0)
    write_out(off, CA, rows0)
    gather_start(TAIL_OFF, CA, rows0, sem0)
    gather_wait(off + CA, CB, rows1, sem1)
    write_out(off + CA, CB, rows1)
    gather_wait(TAIL_OFF, CA, rows0, sem0)
    write_out(TAIL_OFF, CA, rows0)


_emb = functools.partial(
    pl.kernel,
    out_type=jax.ShapeDtypeStruct((N, 8, 128), jnp.float32),
    mesh=plsc.VectorSubcoreMesh(core_axis_name="c", subcore_axis_name="s",
                                num_cores=NC, num_subcores=NS),
    scratch_types=[
        pltpu.VMEM_SHARED((VOCAB, 8, 128), jnp.float32),
        pltpu.VMEM((B_PER_W,), jnp.int32),
        pltpu.VMEM((CA, 8, 128), jnp.float32),
        pltpu.VMEM((CB, 8, 128), jnp.float32),
        pltpu.SemaphoreType.DMA,
        pltpu.SemaphoreType.DMA,
    ],
)(_emb_body)


@jax.jit
def kernel(x, table):
    idx = x.reshape(-1).astype(jnp.int32)
    table_p = jnp.pad(table, ((0, 0), (0, D_PAD - D))).reshape(VOCAB, 8, 128)
    out = _emb(idx, table_p)
    return out.reshape(B, L, D_PAD)[:, :, :D]
